# baseline (speedup 1.0000x reference)
import functools

import jax
import jax.numpy as jnp
from jax.experimental import pallas as pl

_N = 1024
_NT = _N // 8
_KNN_K = 5
_ALPHA = 1.05
_W1 = 5.0
_W2 = 3.0
_BIG = 3.0e38
_DN = (((1,), (1,)), ((), ()))


def _body(m2_ref, apn_ref, aa_ref, aar_ref, m1_ref, oo_ref, out_ref):
    b = pl.program_id(0)
    m2 = m2_ref[0]
    apn = apn_ref[0]
    aa = aa_ref[0]
    aa_row = aar_ref[0]
    m1 = m1_ref[0]
    oo = oo_ref[0]

    inner2 = jax.lax.dot_general(m2, apn, _DN,
                                 preferred_element_type=jnp.float32)
    inner1 = jax.lax.dot_general(m1, apn, _DN,
                                 preferred_element_type=jnp.float32)

    cm = (aa_row + inner1[0:8, :]) + oo[0:8, :]
    R = [jnp.full((8, _N), _BIG, jnp.float32) for _ in range(6)]
    for k in range(_NT):
        x = (aa_row + inner2[k * 8:(k + 1) * 8, :]) + aa[k * 8:(k + 1) * 8, :]
        for j in range(5):
            mj = jnp.minimum(R[j], x)
            x = jnp.maximum(R[j], x)
            R[j] = mj
        R[5] = jnp.minimum(R[5], x)
        if k > 0:
            y = (aa_row + inner1[k * 8:(k + 1) * 8, :]) + oo[k * 8:(k + 1) * 8, :]
            cm = jnp.minimum(cm, y)

    l1 = jnp.mean(jnp.min(cm, axis=0))

    S = jnp.concatenate(R, axis=0)
    row = jax.lax.broadcasted_iota(jnp.int32, (48, _N), 0)
    acc = jnp.zeros((1, _N), jnp.float32)
    for j in range(_KNN_K + 1):
        m = jnp.min(S, axis=0, keepdims=True)
        if j > 0:
            acc = acc + m
        if j < _KNN_K:
            idx = jnp.min(jnp.where(S == m, row, 48), axis=0, keepdims=True)
            S = jnp.where(row == idx, _BIG, S)

    value = acc / jnp.float32(_KNN_K)
    mean = jnp.mean(value)
    std = jnp.sqrt(jnp.sum((value - mean) ** 2) / jnp.float32(_N - 1))
    thr = mean + _ALPHA * std
    w = (value > thr).astype(jnp.float32)
    knn = jnp.mean(value * w)

    part = (_W1 * l1 + _W2 * knn) * jnp.float32(1.0 / 8.0)

    @pl.when(b == 0)
    def _():
        out_ref[...] = jnp.zeros((1, 128), jnp.float32)

    out_ref[...] += jnp.full((1, 128), part, jnp.float32)


@functools.partial(jax.jit, static_argnames=())
def kernel(adv_pc, ori_pc):
    B = adv_pc.shape[0]
    aa = jnp.sum(adv_pc * adv_pc, axis=-1, keepdims=True)
    oo = jnp.sum(ori_pc * ori_pc, axis=-1, keepdims=True)
    zeros = jnp.zeros_like(adv_pc)
    z5 = jnp.concatenate([zeros, zeros[..., :2]], axis=-1)
    m2 = jnp.concatenate([-2.0 * adv_pc, z5], axis=-1)
    apn = jnp.concatenate([adv_pc, z5], axis=-1)
    m1 = jnp.concatenate([-2.0 * ori_pc, z5], axis=-1)
    aar = aa.reshape(B, 1, _N)

    out = pl.pallas_call(
        _body,
        grid=(B,),
        in_specs=[
            pl.BlockSpec((1, _N, 8), lambda b: (b, 0, 0)),
            pl.BlockSpec((1, _N, 8), lambda b: (b, 0, 0)),
            pl.BlockSpec((1, _N, 1), lambda b: (b, 0, 0)),
            pl.BlockSpec((1, 1, _N), lambda b: (b, 0, 0)),
            pl.BlockSpec((1, _N, 8), lambda b: (b, 0, 0)),
            pl.BlockSpec((1, _N, 1), lambda b: (b, 0, 0)),
        ],
        out_specs=pl.BlockSpec((1, 128), lambda b: (0, 0)),
        out_shape=jax.ShapeDtypeStruct((1, 128), jnp.float32),
    )(m2, apn, aa, aar, m1, oo)

    return out[0, 0]
